# baseline (speedup 1.0000x reference)
import dataclasses
import functools

import jax
import jax.numpy as jnp
import numpy as np
from jax import lax
from jax.experimental import pallas as pl
from jax.experimental.pallas import tpu as pltpu
from jax.experimental.pallas import tpu_sc as plsc

N = 10000
E = 320000
D = 128
H = 8
Dh = 16
ED = 16
SCALE = Dh ** -0.5

NC = 2
NS = 16
HC = H // NC
HW = HC * Dh
EPT = E // NS
B = 80
NBLK = EPT // B
NROWBLK = N // 8
ACC_W = 80

_GQ = np.zeros((NC, D, D), np.float32)
_GA = np.zeros((NC, D, D), np.float32)
for _c in range(NC):
    for _j in range(HW):
        _GQ[_c, _c * HW + _j, _j] = 1.0
        _GA[_c, _c * HW + _j, HW + _j] = 1.0

_UN = np.zeros((NC, ACC_W, D), np.float32)
_VD = np.zeros((NC, ACC_W, D), np.float32)
for _c in range(NC):
    for _j in range(HW):
        _UN[_c, _j, _c * HW + _j] = 1.0
    for _h in range(HC):
        _VD[_c, HW + _h, (_c * HC + _h) * Dh:(_c * HC + _h + 1) * Dh] = 1.0


def _ln(y, g, b):
    m = jnp.mean(y, axis=-1, keepdims=True)
    v = jnp.mean((y - m) ** 2, axis=-1, keepdims=True)
    return (y - m) / jnp.sqrt(v + 1e-5) * g + b


def _tables_body(x_ref, wq_ref, bq_ref, m_ref, wkv_ref, bkv_ref,
                 qa_ref, kv_ref):
    x = x_ref[...]
    qs = jnp.dot(x, wq_ref[...], preferred_element_type=jnp.float32) \
        + bq_ref[...]
    qa_ref[0] = jnp.dot(qs, m_ref[0], preferred_element_type=jnp.float32)
    kv_ref[0] = jnp.dot(x, wkv_ref[0],
                        preferred_element_type=jnp.float32) + bkv_ref[0]


_TBLK = 1000


def _build_tables(x, Wqs, bqs, M, WKV, bKV):
    full = lambda s: pl.BlockSpec(s, lambda c, i: tuple(0 for _ in s))
    return pl.pallas_call(
        _tables_body,
        grid=(NC, N // _TBLK),
        in_specs=[
            pl.BlockSpec((_TBLK, D), lambda c, i: (i, 0)),
            full((D, D)), full((1, D)),
            pl.BlockSpec((1, D, D), lambda c, i: (c, 0, 0)),
            pl.BlockSpec((1, D, D), lambda c, i: (c, 0, 0)),
            pl.BlockSpec((1, 1, D), lambda c, i: (c, 0, 0)),
        ],
        out_specs=[
            pl.BlockSpec((1, _TBLK, D), lambda c, i: (c, i, 0)),
            pl.BlockSpec((1, _TBLK, D), lambda c, i: (c, i, 0)),
        ],
        out_shape=[
            jax.ShapeDtypeStruct((NC, N, D), jnp.float32),
            jax.ShapeDtypeStruct((NC, N, D), jnp.float32),
        ],
    )(x, Wqs, bqs, M, WKV, bKV)


def _sc_body(qa_hbm, kv_hbm, dst_hbm, src_hbm, ea_hbm, zeros_hbm, out_hbm,
             dstb0, dstb1, srcb0, srcb1, gdst0, gdst1, qab0, qab1,
             kvb0, kvb1, eab0, eab1, msgb, acc,
             semi0, semi1, semga0, semga1, semgk0, semgk1):
    cid = lax.axis_index("c")
    sid = lax.axis_index("s")
    dstb = (dstb0, dstb1)
    srcb = (srcb0, srcb1)
    gdst = (gdst0, gdst1)
    qab = (qab0, qab1)
    kvb = (kvb0, kvb1)
    eab = (eab0, eab1)
    semi = (semi0, semi1)
    semga = (semga0, semga1)
    semgk = (semgk0, semgk1)

    @pl.loop(0, (NROWBLK + NS - 1) // NS)
    def _zero(j):
        blk = j * NS + sid

        @pl.when(blk < NROWBLK)
        def _():
            row = pl.multiple_of(blk * 8, 8)
            pltpu.sync_copy(zeros_hbm, acc.at[pl.ds(row, 8)])

    plsc.subcore_barrier()

    ebase = sid * EPT
    lane = lax.broadcasted_iota(jnp.int32, (16,), 0)
    coff = lax.broadcast(cid * N, (16,))

    def _base(jj):
        return pl.multiple_of(ebase + jj * B, 8)

    def _issue_idx(jj, s):
        base = _base(jj)
        pltpu.async_copy(dst_hbm.at[pl.ds(base, B)], dstb[s], semi[s])
        pltpu.async_copy(src_hbm.at[pl.ds(base, B)], srcb[s], semi[s])
        pltpu.async_copy(ea_hbm.at[pl.ds(base, B)], eab[s], semi[s])

    def _wait_idx(jj, s):
        base = _base(jj)
        pltpu.make_async_copy(dst_hbm.at[pl.ds(base, B)], dstb[s],
                              semi[s]).wait()
        pltpu.make_async_copy(src_hbm.at[pl.ds(base, B)], srcb[s],
                              semi[s]).wait()
        pltpu.make_async_copy(ea_hbm.at[pl.ds(base, B)], eab[s],
                              semi[s]).wait()

    def _issue_gather(s):
        for k in range(B // 16):
            sl = pl.ds(k * 16, 16)
            gdst[s][sl] = dstb[s][sl] + coff
            srcb[s][sl] = srcb[s][sl] + coff
        pltpu.async_copy(qa_hbm.at[gdst[s]], qab[s], semga[s])
        pltpu.async_copy(kv_hbm.at[srcb[s]], kvb[s], semgk[s])

    def _wait_gather(s):
        pltpu.make_async_copy(qa_hbm.at[gdst[s]], qab[s], semga[s]).wait()
        pltpu.make_async_copy(kv_hbm.at[srcb[s]], kvb[s], semgk[s]).wait()

    def _compute(s):
        @pl.loop(0, B, step=4)
        def _edge(i0):
            for u in range(4):
                i = i0 + u
                eav = eab[s][i, :]
                dvec = jnp.zeros((16,), jnp.float32)
                for h in range(HC):
                    qh = qab[s][i, pl.ds(h * Dh, 16)]
                    ah = qab[s][i, pl.ds(HW + h * Dh, 16)]
                    kh = kvb[s][i, pl.ds(h * Dh, 16)]
                    vh = kvb[s][i, pl.ds(HW + h * Dh, 16)]
                    t = qh * kh + ah * eav
                    s_ = jnp.sum(t)
                    exv = jnp.exp(lax.broadcast(s_, (16,)))
                    msgb[i, pl.ds(h * Dh, 16)] = exv * vh
                    dvec = jnp.where(lane == h, exv, dvec)
                msgb[i, pl.ds(HW, 16)] = dvec

    _issue_idx(0, 0)
    _wait_idx(0, 0)
    _issue_gather(0)
    _issue_idx(1, 1)

    @pl.loop(0, NBLK // 2)
    def _blk(j):
        for p in (0, 1):
            jj = 2 * j + p
            q = 1 - p

            @pl.when(jj + 1 < NBLK)
            def _():
                _wait_idx(jj + 1, q)
                _issue_gather(q)

            _wait_gather(p)
            _compute(p)
            if True:
                pltpu.sync_copy(msgb, acc.at[dstb[p]], add=True)

            @pl.when(jj + 2 < NBLK)
            def _():
                _issue_idx(jj + 2, p)

    plsc.subcore_barrier()

    @pl.loop(0, (NROWBLK + NS - 1) // NS)
    def _dump(j):
        blk = j * NS + sid

        @pl.when(blk < NROWBLK)
        def _():
            row = pl.multiple_of(blk * 8, 8)
            orow = pl.multiple_of(cid * N + blk * 8, 8)
            pltpu.sync_copy(acc.at[pl.ds(row, 8)],
                            out_hbm.at[pl.ds(orow, 8)])


def _run_edges(qa, kv, dst, src, edge_attr, zeros):
    mesh = plsc.VectorSubcoreMesh(core_axis_name="c", subcore_axis_name="s")
    cp = pltpu.CompilerParams(use_tc_tiling_on_sc=False)
    if "needs_layout_passes" in pltpu.CompilerParams.__dataclass_fields__:
        cp = dataclasses.replace(cp, needs_layout_passes=False)
    f = pl.kernel(
        _sc_body,
        out_type=jax.ShapeDtypeStruct((NC * N, ACC_W), jnp.float32),
        mesh=mesh,
        compiler_params=cp,
        scratch_types=(
            [pltpu.VMEM((B,), jnp.int32)] * 6
            + [pltpu.VMEM((B, D), jnp.float32)] * 4
            + [pltpu.VMEM((B, ED), jnp.float32)] * 2
            + [pltpu.VMEM((B, ACC_W), jnp.float32),
               pltpu.VMEM_SHARED((N, ACC_W), jnp.float32)]
            + [pltpu.SemaphoreType.DMA] * 6
        ),
    )
    return f(qa, kv, dst, src, edge_attr, zeros)


def _final_body(p0_ref, p1_ref, x_ref, un_ref, vd_ref, wo_ref, bo_ref,
                w1_ref, b1_ref, w2_ref, b2_ref, g1_ref, be1_ref, g2_ref,
                be2_ref, out_ref):
    p0 = p0_ref[...]
    p1 = p1_ref[...]
    numer = jnp.dot(p0, un_ref[0], preferred_element_type=jnp.float32) \
        + jnp.dot(p1, un_ref[1], preferred_element_type=jnp.float32)
    denr = jnp.dot(p0, vd_ref[0], preferred_element_type=jnp.float32) \
        + jnp.dot(p1, vd_ref[1], preferred_element_type=jnp.float32)
    ao = numer / jnp.maximum(denr, 1e-30)
    y = jnp.dot(ao, wo_ref[...],
                preferred_element_type=jnp.float32) + bo_ref[...]
    x1 = _ln(y + x_ref[...], g1_ref[...], be1_ref[...])
    hh = jnp.dot(x1, w1_ref[...],
                 preferred_element_type=jnp.float32) + b1_ref[...]
    hh = hh * 0.5 * (1.0 + lax.erf(hh * (2.0 ** -0.5)))
    hh = jnp.dot(hh, w2_ref[...],
                 preferred_element_type=jnp.float32) + b2_ref[...]
    out_ref[...] = _ln(hh + x1, g2_ref[...], be2_ref[...])


def _finish(parts, x, UN, VD, Wo, bo, W1, b1, W2, b2, g1, be1, g2, be2):
    full = lambda s: pl.BlockSpec(s, lambda i: tuple(0 for _ in s))
    nblk = N // _TBLK
    return pl.pallas_call(
        _final_body,
        grid=(nblk,),
        in_specs=[
            pl.BlockSpec((_TBLK, ACC_W), lambda i: (i, 0)),
            pl.BlockSpec((_TBLK, ACC_W), lambda i, _n=nblk: (i + _n, 0)),
            pl.BlockSpec((_TBLK, D), lambda i: (i, 0)),
            full((NC, ACC_W, D)), full((NC, ACC_W, D)),
            full((D, D)), full((1, D)),
            full((D, 2 * D)), full((1, 2 * D)),
            full((2 * D, D)), full((1, D)),
            full((1, D)), full((1, D)), full((1, D)), full((1, D)),
        ],
        out_specs=pl.BlockSpec((_TBLK, D), lambda i: (i, 0)),
        out_shape=jax.ShapeDtypeStruct((N, D), jnp.float32),
    )(parts, parts, x, UN, VD, Wo, bo, W1, b1, W2, b2, g1, be1, g2, be2)


def kernel(x, edge_index, edge_attr, Wq, bq, Wk, bk, Wv, bv, We, be,
           Wo, bo, W1, b1, W2, b2, g1, be1, g2, be2):
    ei = edge_index.astype(jnp.int32)
    src = ei[0]
    dst = ei[1]

    blocks = We.reshape(ED, H, Dh).transpose(1, 2, 0)
    Wblk = jax.scipy.linalg.block_diag(*[blocks[h] for h in range(H)])

    gq = jnp.asarray(_GQ)
    ga = jnp.asarray(_GA)
    M = gq + jnp.einsum("ij,cjk->cik", Wblk, ga)
    WKV = (jnp.einsum("ij,cjk->cik", Wk, gq)
           + jnp.einsum("ij,cjk->cik", Wv, ga))
    bKV = (jnp.einsum("j,cjk->ck", bk, gq)
           + jnp.einsum("j,cjk->ck", bv, ga))[:, None, :]

    qa, kv = _build_tables(x, Wq * SCALE, (bq * SCALE).reshape(1, D),
                           M, WKV, bKV)
    zeros = jnp.zeros((8, ACC_W), jnp.float32)
    parts = _run_edges(qa.reshape(NC * N, D), kv.reshape(NC * N, D),
                       dst, src, edge_attr, zeros)
    r1 = lambda v: v.reshape(1, -1)
    x2 = _finish(parts, x, jnp.asarray(_UN), jnp.asarray(_VD),
                 Wo, r1(bo), W1, r1(b1), W2, r1(b2),
                 r1(g1), r1(be1), r1(g2), r1(be2))
    return (x2, edge_attr)


# speedup vs baseline: 3.3218x; 3.3218x over previous
import dataclasses
import functools

import jax
import jax.numpy as jnp
import numpy as np
from jax import lax
from jax.experimental import pallas as pl
from jax.experimental.pallas import tpu as pltpu
from jax.experimental.pallas import tpu_sc as plsc

N = 10000
E = 320000
D = 128
H = 8
Dh = 16
ED = 16
SCALE = Dh ** -0.5

NC = 2
NS = 16
HC = H // NC
HW = HC * Dh
EPT = E // NS
B = 80
NBLK = EPT // B
NROWBLK = N // 8
ACC_W = 80

_GQ = np.zeros((NC, D, D), np.float32)
_GA = np.zeros((NC, D, D), np.float32)
for _c in range(NC):
    for _j in range(HW):
        _GQ[_c, _c * HW + _j, _j] = 1.0
        _GA[_c, _c * HW + _j, HW + _j] = 1.0

_UN = np.zeros((NC, ACC_W, D), np.float32)
_VD = np.zeros((NC, ACC_W, D), np.float32)
for _c in range(NC):
    for _j in range(HW):
        _UN[_c, _j, _c * HW + _j] = 1.0
    for _h in range(HC):
        _VD[_c, HW + _h, (_c * HC + _h) * Dh:(_c * HC + _h + 1) * Dh] = 1.0


def _ln(y, g, b):
    m = jnp.mean(y, axis=-1, keepdims=True)
    v = jnp.mean((y - m) ** 2, axis=-1, keepdims=True)
    return (y - m) / jnp.sqrt(v + 1e-5) * g + b


def _tables_body(x_ref, wq_ref, bq_ref, m_ref, wkv_ref, bkv_ref,
                 qa_ref, kv_ref):
    x = x_ref[...]
    qs = jnp.dot(x, wq_ref[...], preferred_element_type=jnp.float32) \
        + bq_ref[...]
    qa_ref[0] = jnp.dot(qs, m_ref[0], preferred_element_type=jnp.float32)
    kv_ref[0] = jnp.dot(x, wkv_ref[0],
                        preferred_element_type=jnp.float32) + bkv_ref[0]


_TBLK = 1000


def _build_tables(x, Wqs, bqs, M, WKV, bKV):
    full = lambda s: pl.BlockSpec(s, lambda c, i: tuple(0 for _ in s))
    return pl.pallas_call(
        _tables_body,
        grid=(NC, N // _TBLK),
        in_specs=[
            pl.BlockSpec((_TBLK, D), lambda c, i: (i, 0)),
            full((D, D)), full((1, D)),
            pl.BlockSpec((1, D, D), lambda c, i: (c, 0, 0)),
            pl.BlockSpec((1, D, D), lambda c, i: (c, 0, 0)),
            pl.BlockSpec((1, 1, D), lambda c, i: (c, 0, 0)),
        ],
        out_specs=[
            pl.BlockSpec((1, _TBLK, D), lambda c, i: (c, i, 0)),
            pl.BlockSpec((1, _TBLK, D), lambda c, i: (c, i, 0)),
        ],
        out_shape=[
            jax.ShapeDtypeStruct((NC, N, D), jnp.float32),
            jax.ShapeDtypeStruct((NC, N, D), jnp.float32),
        ],
    )(x, Wqs, bqs, M, WKV, bKV)


def _sc_body(qa_hbm, kv_hbm, dst_hbm, src_hbm, ea_hbm, zeros_hbm, out_hbm,
             dstb0, dstb1, srcb0, srcb1, gdst0, gdst1, qab0, qab1,
             kvb0, kvb1, eab0, eab1, msgb, acc,
             semi0, semi1, semga0, semga1, semgk0, semgk1):
    cid = lax.axis_index("c")
    sid = lax.axis_index("s")
    dstb = (dstb0, dstb1)
    srcb = (srcb0, srcb1)
    gdst = (gdst0, gdst1)
    qab = (qab0, qab1)
    kvb = (kvb0, kvb1)
    eab = (eab0, eab1)
    semi = (semi0, semi1)
    semga = (semga0, semga1)
    semgk = (semgk0, semgk1)

    @pl.loop(0, (NROWBLK + NS - 1) // NS)
    def _zero(j):
        blk = j * NS + sid

        @pl.when(blk < NROWBLK)
        def _():
            row = pl.multiple_of(blk * 8, 8)
            pltpu.sync_copy(zeros_hbm, acc.at[pl.ds(row, 8)])

    plsc.subcore_barrier()

    ebase = sid * EPT
    lane = lax.broadcasted_iota(jnp.int32, (16,), 0)
    coff = lax.broadcast(cid * N, (16,))

    def _base(jj):
        return pl.multiple_of(ebase + jj * B, 8)

    def _issue_idx(jj, s):
        base = _base(jj)
        pltpu.async_copy(dst_hbm.at[pl.ds(base, B)], dstb[s], semi[s])
        pltpu.async_copy(src_hbm.at[pl.ds(base, B)], srcb[s], semi[s])
        pltpu.async_copy(ea_hbm.at[pl.ds(base, B)], eab[s], semi[s])

    def _wait_idx(jj, s):
        base = _base(jj)
        pltpu.make_async_copy(dst_hbm.at[pl.ds(base, B)], dstb[s],
                              semi[s]).wait()
        pltpu.make_async_copy(src_hbm.at[pl.ds(base, B)], srcb[s],
                              semi[s]).wait()
        pltpu.make_async_copy(ea_hbm.at[pl.ds(base, B)], eab[s],
                              semi[s]).wait()

    def _issue_gather(s):
        for k in range(B // 16):
            sl = pl.ds(k * 16, 16)
            gdst[s][sl] = dstb[s][sl] + coff
            srcb[s][sl] = srcb[s][sl] + coff
        pltpu.async_copy(qa_hbm.at[gdst[s]], qab[s], semga[s])
        pltpu.async_copy(kv_hbm.at[srcb[s]], kvb[s], semgk[s])

    def _wait_gather(s):
        pltpu.make_async_copy(qa_hbm.at[gdst[s]], qab[s], semga[s]).wait()
        pltpu.make_async_copy(kv_hbm.at[srcb[s]], kvb[s], semgk[s]).wait()

    def _compute(s):
        @functools.partial(plsc.parallel_loop, 0, B, unroll=4)
        def _edge(i):
            eav = eab[s][i, :]
            dvec = jnp.zeros((16,), jnp.float32)
            for h in range(HC):
                qh = qab[s][i, pl.ds(h * Dh, 16)]
                ah = qab[s][i, pl.ds(HW + h * Dh, 16)]
                kh = kvb[s][i, pl.ds(h * Dh, 16)]
                vh = kvb[s][i, pl.ds(HW + h * Dh, 16)]
                t = qh * kh + ah * eav
                s_ = jnp.sum(t)
                exv = jnp.exp(lax.broadcast(s_, (16,)))
                msgb[i, pl.ds(h * Dh, 16)] = exv * vh
                dvec = jnp.where(lane == h, exv, dvec)
            msgb[i, pl.ds(HW, 16)] = dvec

    _issue_idx(0, 0)
    _wait_idx(0, 0)
    _issue_gather(0)
    _issue_idx(1, 1)

    @pl.loop(0, NBLK // 2)
    def _blk(j):
        for p in (0, 1):
            jj = 2 * j + p
            q = 1 - p

            @pl.when(jj + 1 < NBLK)
            def _():
                _wait_idx(jj + 1, q)
                _issue_gather(q)

            _wait_gather(p)
            _compute(p)
            if True:
                pltpu.sync_copy(msgb, acc.at[dstb[p]], add=True)

            @pl.when(jj + 2 < NBLK)
            def _():
                _issue_idx(jj + 2, p)

    plsc.subcore_barrier()

    @pl.loop(0, (NROWBLK + NS - 1) // NS)
    def _dump(j):
        blk = j * NS + sid

        @pl.when(blk < NROWBLK)
        def _():
            row = pl.multiple_of(blk * 8, 8)
            orow = pl.multiple_of(cid * N + blk * 8, 8)
            pltpu.sync_copy(acc.at[pl.ds(row, 8)],
                            out_hbm.at[pl.ds(orow, 8)])


def _run_edges(qa, kv, dst, src, edge_attr, zeros):
    mesh = plsc.VectorSubcoreMesh(core_axis_name="c", subcore_axis_name="s")
    cp = pltpu.CompilerParams(use_tc_tiling_on_sc=False)
    if "needs_layout_passes" in pltpu.CompilerParams.__dataclass_fields__:
        cp = dataclasses.replace(cp, needs_layout_passes=False)
    f = pl.kernel(
        _sc_body,
        out_type=jax.ShapeDtypeStruct((NC * N, ACC_W), jnp.float32),
        mesh=mesh,
        compiler_params=cp,
        scratch_types=(
            [pltpu.VMEM((B,), jnp.int32)] * 6
            + [pltpu.VMEM((B, D), jnp.float32)] * 4
            + [pltpu.VMEM((B, ED), jnp.float32)] * 2
            + [pltpu.VMEM((B, ACC_W), jnp.float32),
               pltpu.VMEM_SHARED((N, ACC_W), jnp.float32)]
            + [pltpu.SemaphoreType.DMA] * 6
        ),
    )
    return f(qa, kv, dst, src, edge_attr, zeros)


def _final_body(p0_ref, p1_ref, x_ref, un_ref, vd_ref, wo_ref, bo_ref,
                w1_ref, b1_ref, w2_ref, b2_ref, g1_ref, be1_ref, g2_ref,
                be2_ref, out_ref):
    p0 = p0_ref[...]
    p1 = p1_ref[...]
    numer = jnp.dot(p0, un_ref[0], preferred_element_type=jnp.float32) \
        + jnp.dot(p1, un_ref[1], preferred_element_type=jnp.float32)
    denr = jnp.dot(p0, vd_ref[0], preferred_element_type=jnp.float32) \
        + jnp.dot(p1, vd_ref[1], preferred_element_type=jnp.float32)
    ao = numer / jnp.maximum(denr, 1e-30)
    y = jnp.dot(ao, wo_ref[...],
                preferred_element_type=jnp.float32) + bo_ref[...]
    x1 = _ln(y + x_ref[...], g1_ref[...], be1_ref[...])
    hh = jnp.dot(x1, w1_ref[...],
                 preferred_element_type=jnp.float32) + b1_ref[...]
    hh = hh * 0.5 * (1.0 + lax.erf(hh * (2.0 ** -0.5)))
    hh = jnp.dot(hh, w2_ref[...],
                 preferred_element_type=jnp.float32) + b2_ref[...]
    out_ref[...] = _ln(hh + x1, g2_ref[...], be2_ref[...])


def _finish(parts, x, UN, VD, Wo, bo, W1, b1, W2, b2, g1, be1, g2, be2):
    full = lambda s: pl.BlockSpec(s, lambda i: tuple(0 for _ in s))
    nblk = N // _TBLK
    return pl.pallas_call(
        _final_body,
        grid=(nblk,),
        in_specs=[
            pl.BlockSpec((_TBLK, ACC_W), lambda i: (i, 0)),
            pl.BlockSpec((_TBLK, ACC_W), lambda i, _n=nblk: (i + _n, 0)),
            pl.BlockSpec((_TBLK, D), lambda i: (i, 0)),
            full((NC, ACC_W, D)), full((NC, ACC_W, D)),
            full((D, D)), full((1, D)),
            full((D, 2 * D)), full((1, 2 * D)),
            full((2 * D, D)), full((1, D)),
            full((1, D)), full((1, D)), full((1, D)), full((1, D)),
        ],
        out_specs=pl.BlockSpec((_TBLK, D), lambda i: (i, 0)),
        out_shape=jax.ShapeDtypeStruct((N, D), jnp.float32),
    )(parts, parts, x, UN, VD, Wo, bo, W1, b1, W2, b2, g1, be1, g2, be2)


def kernel(x, edge_index, edge_attr, Wq, bq, Wk, bk, Wv, bv, We, be,
           Wo, bo, W1, b1, W2, b2, g1, be1, g2, be2):
    ei = edge_index.astype(jnp.int32)
    src = ei[0]
    dst = ei[1]

    blocks = We.reshape(ED, H, Dh).transpose(1, 2, 0)
    Wblk = jax.scipy.linalg.block_diag(*[blocks[h] for h in range(H)])

    gq = jnp.asarray(_GQ)
    ga = jnp.asarray(_GA)
    M = gq + jnp.einsum("ij,cjk->cik", Wblk, ga)
    WKV = (jnp.einsum("ij,cjk->cik", Wk, gq)
           + jnp.einsum("ij,cjk->cik", Wv, ga))
    bKV = (jnp.einsum("j,cjk->ck", bk, gq)
           + jnp.einsum("j,cjk->ck", bv, ga))[:, None, :]

    qa, kv = _build_tables(x, Wq * SCALE, (bq * SCALE).reshape(1, D),
                           M, WKV, bKV)
    zeros = jnp.zeros((8, ACC_W), jnp.float32)
    parts = _run_edges(qa.reshape(NC * N, D), kv.reshape(NC * N, D),
                       dst, src, edge_attr, zeros)
    r1 = lambda v: v.reshape(1, -1)
    x2 = _finish(parts, x, jnp.asarray(_UN), jnp.asarray(_VD),
                 Wo, r1(bo), W1, r1(b1), W2, r1(b2),
                 r1(g1), r1(be1), r1(g2), r1(be2))
    return (x2, edge_attr)
